# baseline (device time: 14346 ns/iter reference)
import jax
import jax.numpy as jnp
from jax import lax
from jax.experimental import pallas as pl
from jax.experimental.pallas import tpu as pltpu

N_DEV = 4
B = 2
SQ = 256
SKV = 256
HQ = 4
DH = 64
D = HQ * DH
HALF = 128
NEG = -1e9
DL = D + 8
BF = jnp.bfloat16


def kernel(x, Wq, K_ext, V_ext, Wo):

    def body(x_ref, wq_ref, k_ref, v_ref, wo_ref, out_ref,
             o_buf, ctx_buf, k2, v2, kv_sems,
             ex_send, ex_recv, cx_send, cx_recv):
        my_pos = lax.axis_index("i")
        left = lax.rem(my_pos + (N_DEV - 1), N_DEV)
        right = lax.rem(my_pos + 1, N_DEV)
        is01 = my_pos <= 1

        barrier_sem = pltpu.get_barrier_semaphore()
        for nbr in (left, right):
            pl.semaphore_signal(
                barrier_sem, inc=1,
                device_id=(nbr,), device_id_type=pl.DeviceIdType.MESH,
            )
        pl.semaphore_wait(barrier_sem, 2)

        def proj(rows, bs=None):
            n = rows.stop - rows.start
            if bs is None:
                c = ctx_buf[:, rows, :].reshape(B * n, D)
                o = jnp.dot(c, wo_ref[...].astype(BF),
                            preferred_element_type=jnp.float32)
                out_ref[:, rows, :] = o.reshape(B, n, 512)
            else:
                c = ctx_buf[bs, rows, :]
                o = jnp.dot(c, wo_ref[...].astype(BF),
                            preferred_element_type=jnp.float32)
                out_ref[bs, rows, :] = o

        def attn_block(qrows, kbh, vbh, bias):
            s = lax.dot_general(
                qrows, kbh, (((1,), (1,)), ((), ())),
                preferred_element_type=jnp.float32,
            )
            w = jnp.exp((s + bias).astype(BF))
            l = jnp.sum(w, axis=1, keepdims=True, dtype=jnp.float32)
            o = jnp.dot(w, vbh, preferred_element_type=jnp.float32)
            return o, l

        def combine_hi(b):
            p = o_buf[0, b, HALF:, :] + o_buf[1, b, HALF:, :]
            for h in range(HQ):
                hs = slice(h * DH, (h + 1) * DH)
                ctx_buf[b, HALF:, hs] = p[:, hs] / p[:, D + h:D + h + 1]

        @pl.when(is01)
        def _producer():
            partner = 1 - my_pos
            rd_ex = [
                pltpu.make_async_remote_copy(
                    src_ref=o_buf.at[0, b, pl.ds(HALF, HALF), :],
                    dst_ref=o_buf.at[1, b, pl.ds(HALF, HALF), :],
                    send_sem=ex_send.at[b], recv_sem=ex_recv.at[b],
                    device_id=(partner,),
                    device_id_type=pl.DeviceIdType.MESH,
                )
                for b in range(B)
            ]
            rd_lo3 = [
                pltpu.make_async_remote_copy(
                    src_ref=ctx_buf.at[b, pl.ds(0, HALF), :],
                    dst_ref=ctx_buf.at[b, pl.ds(0, HALF), :],
                    send_sem=cx_send.at[b], recv_sem=cx_recv.at[b],
                    device_id=(3,), device_id_type=pl.DeviceIdType.MESH,
                )
                for b in range(B)
            ]
            rd_lo1 = [
                pltpu.make_async_remote_copy(
                    src_ref=ctx_buf.at[b, pl.ds(0, HALF), :],
                    dst_ref=ctx_buf.at[b, pl.ds(0, HALF), :],
                    send_sem=cx_send.at[2 + b], recv_sem=cx_recv.at[b],
                    device_id=(1,), device_id_type=pl.DeviceIdType.MESH,
                )
                for b in range(B)
            ]

            @pl.when(my_pos == 0)
            def _p0():
                dmas = [
                    [
                        pltpu.make_async_copy(
                            src.at[b, :, h, :],
                            dst.at[b, h],
                            kv_sems.at[a, b, h],
                        )
                        for a, (src, dst) in enumerate(
                            [(k_ref, k2), (v_ref, v2)])
                        for h in range(HQ)
                    ]
                    for b in range(B)
                ]
                for b in range(B):
                    for d in dmas[b]:
                        d.start()
                ti = lax.broadcasted_iota(jnp.int32, (HALF, SKV), 0)
                tj = lax.broadcasted_iota(jnp.int32, (HALF, SKV), 1)
                bias = jnp.where(tj >= ti, 0.0, NEG)
                xhi = x_ref[:, HALF:, :].reshape(B * HALF, 512).astype(BF)
                qhi = (jnp.dot(xhi, wq_ref[...].astype(BF),
                               preferred_element_type=jnp.float32)
                       * 0.125).astype(BF)
                for b in range(B):
                    for d in dmas[b]:
                        d.wait()
                    for h in range(HQ):
                        hs = slice(h * DH, (h + 1) * DH)
                        o, l = attn_block(
                            qhi[b * HALF:(b + 1) * HALF, hs],
                            k2[b, h].astype(BF), v2[b, h].astype(BF), bias)
                        o_buf[0, b, HALF:, hs] = o.astype(BF)
                        o_buf[0, b, HALF:, D + h:D + h + 1] = l.astype(BF)
                    rd_ex[b].start()

                bias_lo = jnp.where(tj - ti <= 128, 0.0, NEG)
                xlo = x_ref[:, 0:HALF, :].reshape(B * HALF, 512).astype(BF)
                qlo = (jnp.dot(xlo, wq_ref[...].astype(BF),
                               preferred_element_type=jnp.float32)
                       * 0.125).astype(BF)
                for b in range(B):
                    for h in range(HQ):
                        hs = slice(h * DH, (h + 1) * DH)
                        o, l = attn_block(
                            qlo[b * HALF:(b + 1) * HALF, hs],
                            k2[b, h].astype(BF), v2[b, h].astype(BF),
                            bias_lo)
                        ctx_buf[b, 0:HALF, hs] = (o / l).astype(BF)
                    rd_lo3[b].start()
                    rd_lo1[b].start()

                rd_hi3 = [
                    pltpu.make_async_remote_copy(
                        src_ref=ctx_buf.at[b, pl.ds(HALF, HALF), :],
                        dst_ref=ctx_buf.at[b, pl.ds(HALF, HALF), :],
                        send_sem=cx_send.at[4 + b], recv_sem=cx_recv.at[2 + b],
                        device_id=(3,), device_id_type=pl.DeviceIdType.MESH,
                    )
                    for b in range(B)
                ]
                for b in range(B):
                    rd_ex[b].wait()
                    combine_hi(b)
                    rd_hi3[b].start()
                proj(slice(0, SQ))
                for b in range(B):
                    rd_lo3[b].wait_send()
                    rd_lo1[b].wait_send()
                    rd_hi3[b].wait_send()

            @pl.when(my_pos == 1)
            def _p1():
                dmas = [
                    [
                        pltpu.make_async_copy(
                            src.at[b, pl.ds(0, HALF), h, :],
                            dst.at[b, h, pl.ds(0, HALF), :],
                            kv_sems.at[a, b, h],
                        )
                        for a, (src, dst) in enumerate(
                            [(k_ref, k2), (v_ref, v2)])
                        for h in range(HQ)
                    ]
                    for b in range(B)
                ]
                for b in range(B):
                    for d in dmas[b]:
                        d.start()
                ti = lax.broadcasted_iota(jnp.int32, (HALF, HALF), 0)
                tj = lax.broadcasted_iota(jnp.int32, (HALF, HALF), 1)
                bias = jnp.where(tj <= ti, 0.0, NEG)
                xhi = x_ref[:, HALF:, :].reshape(B * HALF, 512).astype(BF)
                qhi = (jnp.dot(xhi, wq_ref[...].astype(BF),
                               preferred_element_type=jnp.float32)
                       * 0.125).astype(BF)
                for b in range(B):
                    for d in dmas[b]:
                        d.wait()
                    for h in range(HQ):
                        hs = slice(h * DH, (h + 1) * DH)
                        o, l = attn_block(
                            qhi[b * HALF:(b + 1) * HALF, hs],
                            k2[b, h, 0:HALF, :].astype(BF),
                            v2[b, h, 0:HALF, :].astype(BF), bias)
                        o_buf[0, b, HALF:, hs] = o.astype(BF)
                        o_buf[0, b, HALF:, D + h:D + h + 1] = l.astype(BF)
                    rd_ex[b].start()

                rd_hi2 = [
                    pltpu.make_async_remote_copy(
                        src_ref=ctx_buf.at[b, pl.ds(HALF, HALF), :],
                        dst_ref=ctx_buf.at[b, pl.ds(HALF, HALF), :],
                        send_sem=cx_send.at[b],
                        recv_sem=cx_recv.at[2 + b],
                        device_id=(2,), device_id_type=pl.DeviceIdType.MESH,
                    )
                    for b in range(B)
                ]
                for b in range(B):
                    rd_ex[b].wait()
                    combine_hi(b)
                    rd_hi2[b].start()
                    proj(slice(HALF, SQ), bs=b)
                for b in range(B):
                    rd_lo1[b].wait_recv()
                    proj(slice(0, HALF), bs=b)
                rd_hi2[0].wait_send()
                rd_hi2[1].wait_send()

        @pl.when(jnp.logical_not(is01))
        def _consumer():
            rd_lo = [
                pltpu.make_async_remote_copy(
                    src_ref=ctx_buf.at[b, pl.ds(0, HALF), :],
                    dst_ref=ctx_buf.at[b, pl.ds(0, HALF), :],
                    send_sem=cx_send.at[b], recv_sem=cx_recv.at[b],
                    device_id=(left,), device_id_type=pl.DeviceIdType.MESH,
                )
                for b in range(B)
            ]

            @pl.when(my_pos == 3)
            def _c3():
                rd_hi = [
                    pltpu.make_async_remote_copy(
                        src_ref=ctx_buf.at[b, pl.ds(HALF, HALF), :],
                        dst_ref=ctx_buf.at[b, pl.ds(HALF, HALF), :],
                        send_sem=cx_send.at[4 + b], recv_sem=cx_recv.at[2 + b],
                        device_id=(0,), device_id_type=pl.DeviceIdType.MESH,
                    )
                    for b in range(B)
                ]
                rd_fwd = [
                    pltpu.make_async_remote_copy(
                        src_ref=ctx_buf.at[b, pl.ds(0, HALF), :],
                        dst_ref=ctx_buf.at[b, pl.ds(0, HALF), :],
                        send_sem=cx_send.at[b], recv_sem=cx_recv.at[b],
                        device_id=(2,), device_id_type=pl.DeviceIdType.MESH,
                    )
                    for b in range(B)
                ]
                for b in range(B):
                    rd_lo[b].wait_recv()
                    rd_fwd[b].start()
                    proj(slice(0, HALF), bs=b)
                for b in range(B):
                    rd_hi[b].wait_recv()
                    proj(slice(HALF, SQ), bs=b)
                rd_fwd[0].wait_send()
                rd_fwd[1].wait_send()

            @pl.when(my_pos == 2)
            def _c2():
                rd_hi2 = [
                    pltpu.make_async_remote_copy(
                        src_ref=ctx_buf.at[b, pl.ds(HALF, HALF), :],
                        dst_ref=ctx_buf.at[b, pl.ds(HALF, HALF), :],
                        send_sem=cx_send.at[b],
                        recv_sem=cx_recv.at[2 + b],
                        device_id=(1,), device_id_type=pl.DeviceIdType.MESH,
                    )
                    for b in range(B)
                ]
                rd_hi2[0].wait_recv()
                proj(slice(HALF, SQ), bs=0)
                rd_hi2[1].wait_recv()
                proj(slice(HALF, SQ), bs=1)
                rd_lo[0].wait_recv()
                proj(slice(0, HALF), bs=0)
                rd_lo[1].wait_recv()
                proj(slice(0, HALF), bs=1)

    return pl.pallas_call(
        body,
        out_shape=jax.ShapeDtypeStruct((B, SQ, 512), jnp.float32),
        in_specs=[
            pl.BlockSpec(memory_space=pltpu.VMEM),
            pl.BlockSpec(memory_space=pltpu.VMEM),
            pl.BlockSpec(memory_space=pltpu.MemorySpace.HBM),
            pl.BlockSpec(memory_space=pltpu.MemorySpace.HBM),
            pl.BlockSpec(memory_space=pltpu.VMEM),
        ],
        out_specs=pl.BlockSpec(memory_space=pltpu.VMEM),
        scratch_shapes=[
            pltpu.VMEM((2, B, SQ, DL), BF),
            pltpu.VMEM((B, SQ, D), BF),
            pltpu.VMEM((B, HQ, SKV, DH), jnp.float32),
            pltpu.VMEM((B, HQ, SKV, DH), jnp.float32),
            pltpu.SemaphoreType.DMA((2, B, HQ)),
            pltpu.SemaphoreType.DMA((B,)),
            pltpu.SemaphoreType.DMA((B,)),
            pltpu.SemaphoreType.DMA((6,)),
            pltpu.SemaphoreType.DMA((4,)),
        ],
        compiler_params=pltpu.CompilerParams(collective_id=0),
    )(x, Wq, K_ext, V_ext, Wo)


# device time: 13094 ns/iter; 1.0956x vs baseline; 1.0956x over previous
import jax
import jax.numpy as jnp
from jax import lax
from jax.experimental import pallas as pl
from jax.experimental.pallas import tpu as pltpu

N_DEV = 4
B = 2
SQ = 256
SKV = 256
HQ = 4
DH = 64
D = HQ * DH
HALF = 128
NEG = -1e9
DL = D + 8
BF = jnp.bfloat16


def kernel(x, Wq, K_ext, V_ext, Wo):
    K2 = K_ext.reshape(B, SKV, D).astype(BF)
    V2 = V_ext.reshape(B, SKV, D).astype(BF)

    def body(x_ref, wq_ref, k_ref, v_ref, wo_ref, out_ref,
             o_buf, ctx_buf, out_v, out_sems,
             ex_send, ex_recv, cx_send, cx_recv):
        my_pos = lax.axis_index("i")
        left = lax.rem(my_pos + (N_DEV - 1), N_DEV)
        right = lax.rem(my_pos + 1, N_DEV)
        is01 = my_pos <= 1

        barrier_sem = pltpu.get_barrier_semaphore()
        for nbr in (left, right):
            pl.semaphore_signal(
                barrier_sem, inc=1,
                device_id=(nbr,), device_id_type=pl.DeviceIdType.MESH,
            )
        pl.semaphore_wait(barrier_sem, 2)

        def proj(rows, bs=None):
            n = rows.stop - rows.start
            if bs is None:
                c = ctx_buf[:, rows, :].reshape(B * n, D)
                o = jnp.dot(c, wo_ref[...].astype(BF),
                            preferred_element_type=jnp.float32)
                out_v[:, rows, :] = o.reshape(B, n, 512)
            else:
                c = ctx_buf[bs, rows, :]
                o = jnp.dot(c, wo_ref[...].astype(BF),
                            preferred_element_type=jnp.float32)
                out_v[bs, rows, :] = o

        def out_dma(i, rows, bs=None):
            if bs is None:
                cp = pltpu.make_async_copy(
                    out_v.at[:, rows, :], out_ref.at[:, rows, :],
                    out_sems.at[i])
            else:
                cp = pltpu.make_async_copy(
                    out_v.at[bs, rows, :], out_ref.at[bs, rows, :],
                    out_sems.at[i])
            cp.start()
            return cp

        def attn_block(qrows, kbh, vbh, bias):
            s = lax.dot_general(
                qrows, kbh, (((1,), (1,)), ((), ())),
                preferred_element_type=jnp.float32,
            )
            w = jnp.exp((s + bias).astype(BF))
            l = jnp.sum(w, axis=1, keepdims=True, dtype=jnp.float32)
            o = jnp.dot(w, vbh, preferred_element_type=jnp.float32)
            return o, l

        def combine_hi(b):
            p = o_buf[0, b, HALF:, :] + o_buf[1, b, HALF:, :]
            for h in range(HQ):
                hs = slice(h * DH, (h + 1) * DH)
                ctx_buf[b, HALF:, hs] = p[:, hs] / p[:, D + h:D + h + 1]

        @pl.when(is01)
        def _producer():
            partner = 1 - my_pos
            rd_ex = [
                pltpu.make_async_remote_copy(
                    src_ref=o_buf.at[0, b, pl.ds(HALF, HALF), :],
                    dst_ref=o_buf.at[1, b, pl.ds(HALF, HALF), :],
                    send_sem=ex_send.at[b], recv_sem=ex_recv.at[b],
                    device_id=(partner,),
                    device_id_type=pl.DeviceIdType.MESH,
                )
                for b in range(B)
            ]
            rd_lo3 = [
                pltpu.make_async_remote_copy(
                    src_ref=ctx_buf.at[b, pl.ds(0, HALF), :],
                    dst_ref=ctx_buf.at[b, pl.ds(0, HALF), :],
                    send_sem=cx_send.at[b], recv_sem=cx_recv.at[b],
                    device_id=(3,), device_id_type=pl.DeviceIdType.MESH,
                )
                for b in range(B)
            ]
            rd_lo1 = [
                pltpu.make_async_remote_copy(
                    src_ref=ctx_buf.at[b, pl.ds(0, HALF), :],
                    dst_ref=ctx_buf.at[b, pl.ds(0, HALF), :],
                    send_sem=cx_send.at[2 + b], recv_sem=cx_recv.at[b],
                    device_id=(1,), device_id_type=pl.DeviceIdType.MESH,
                )
                for b in range(B)
            ]

            @pl.when(my_pos == 0)
            def _p0():
                ti = lax.broadcasted_iota(jnp.int32, (HALF, SKV), 0)
                tj = lax.broadcasted_iota(jnp.int32, (HALF, SKV), 1)
                bias = jnp.where(tj >= ti, 0.0, NEG)
                xhi = x_ref[:, HALF:, :].reshape(B * HALF, 512).astype(BF)
                qhi = (jnp.dot(xhi, wq_ref[...].astype(BF),
                               preferred_element_type=jnp.float32)
                       * 0.125).astype(BF)
                for b in range(B):
                    kb = k_ref[b]
                    vb = v_ref[b]
                    for h in range(HQ):
                        hs = slice(h * DH, (h + 1) * DH)
                        o, l = attn_block(
                            qhi[b * HALF:(b + 1) * HALF, hs],
                            kb[:, hs], vb[:, hs], bias)
                        o_buf[0, b, HALF:, hs] = o.astype(BF)
                        o_buf[0, b, HALF:, D + h:D + h + 1] = l.astype(BF)
                    rd_ex[b].start()

                bias_lo = jnp.where(tj - ti <= 128, 0.0, NEG)
                xlo = x_ref[:, 0:HALF, :].reshape(B * HALF, 512).astype(BF)
                qlo = (jnp.dot(xlo, wq_ref[...].astype(BF),
                               preferred_element_type=jnp.float32)
                       * 0.125).astype(BF)
                for b in range(B):
                    kb = k_ref[b]
                    vb = v_ref[b]
                    for h in range(HQ):
                        hs = slice(h * DH, (h + 1) * DH)
                        o, l = attn_block(
                            qlo[b * HALF:(b + 1) * HALF, hs],
                            kb[:, hs], vb[:, hs], bias_lo)
                        ctx_buf[b, 0:HALF, hs] = (o / l).astype(BF)
                    rd_lo3[b].start()
                    rd_lo1[b].start()

                rd_hi3 = [
                    pltpu.make_async_remote_copy(
                        src_ref=ctx_buf.at[b, pl.ds(HALF, HALF), :],
                        dst_ref=ctx_buf.at[b, pl.ds(HALF, HALF), :],
                        send_sem=cx_send.at[4 + b], recv_sem=cx_recv.at[2 + b],
                        device_id=(3,), device_id_type=pl.DeviceIdType.MESH,
                    )
                    for b in range(B)
                ]
                for b in range(B):
                    rd_ex[b].wait()
                    combine_hi(b)
                    rd_hi3[b].start()
                proj(slice(0, SQ))
                cp = out_dma(0, slice(0, SQ))
                for b in range(B):
                    rd_lo3[b].wait_send()
                    rd_lo1[b].wait_send()
                    rd_hi3[b].wait_send()
                cp.wait()

            @pl.when(my_pos == 1)
            def _p1():
                ti = lax.broadcasted_iota(jnp.int32, (HALF, HALF), 0)
                tj = lax.broadcasted_iota(jnp.int32, (HALF, HALF), 1)
                bias = jnp.where(tj <= ti, 0.0, NEG)
                xhi = x_ref[:, HALF:, :].reshape(B * HALF, 512).astype(BF)
                qhi = (jnp.dot(xhi, wq_ref[...].astype(BF),
                               preferred_element_type=jnp.float32)
                       * 0.125).astype(BF)
                for b in range(B):
                    kb = k_ref[b, 0:HALF, :]
                    vb = v_ref[b, 0:HALF, :]
                    for h in range(HQ):
                        hs = slice(h * DH, (h + 1) * DH)
                        o, l = attn_block(
                            qhi[b * HALF:(b + 1) * HALF, hs],
                            kb[:, hs], vb[:, hs], bias)
                        o_buf[0, b, HALF:, hs] = o.astype(BF)
                        o_buf[0, b, HALF:, D + h:D + h + 1] = l.astype(BF)
                    rd_ex[b].start()

                rd_hi2 = [
                    pltpu.make_async_remote_copy(
                        src_ref=ctx_buf.at[b, pl.ds(HALF, HALF), :],
                        dst_ref=ctx_buf.at[b, pl.ds(HALF, HALF), :],
                        send_sem=cx_send.at[b],
                        recv_sem=cx_recv.at[2 + b],
                        device_id=(2,), device_id_type=pl.DeviceIdType.MESH,
                    )
                    for b in range(B)
                ]
                cps = []
                for b in range(B):
                    rd_ex[b].wait()
                    combine_hi(b)
                    rd_hi2[b].start()
                    proj(slice(HALF, SQ), bs=b)
                    cps.append(out_dma(b, slice(HALF, SQ), bs=b))
                for b in range(B):
                    rd_lo1[b].wait_recv()
                    proj(slice(0, HALF), bs=b)
                    cps.append(out_dma(2 + b, slice(0, HALF), bs=b))
                rd_hi2[0].wait_send()
                rd_hi2[1].wait_send()
                for cp in cps:
                    cp.wait()

        @pl.when(jnp.logical_not(is01))
        def _consumer():
            rd_lo = [
                pltpu.make_async_remote_copy(
                    src_ref=ctx_buf.at[b, pl.ds(0, HALF), :],
                    dst_ref=ctx_buf.at[b, pl.ds(0, HALF), :],
                    send_sem=cx_send.at[b], recv_sem=cx_recv.at[b],
                    device_id=(left,), device_id_type=pl.DeviceIdType.MESH,
                )
                for b in range(B)
            ]

            @pl.when(my_pos == 3)
            def _c3():
                rd_hi = [
                    pltpu.make_async_remote_copy(
                        src_ref=ctx_buf.at[b, pl.ds(HALF, HALF), :],
                        dst_ref=ctx_buf.at[b, pl.ds(HALF, HALF), :],
                        send_sem=cx_send.at[4 + b], recv_sem=cx_recv.at[2 + b],
                        device_id=(0,), device_id_type=pl.DeviceIdType.MESH,
                    )
                    for b in range(B)
                ]
                rd_fwd = [
                    pltpu.make_async_remote_copy(
                        src_ref=ctx_buf.at[b, pl.ds(0, HALF), :],
                        dst_ref=ctx_buf.at[b, pl.ds(0, HALF), :],
                        send_sem=cx_send.at[b], recv_sem=cx_recv.at[b],
                        device_id=(2,), device_id_type=pl.DeviceIdType.MESH,
                    )
                    for b in range(B)
                ]
                cps = []
                for b in range(B):
                    rd_lo[b].wait_recv()
                    rd_fwd[b].start()
                    proj(slice(0, HALF), bs=b)
                    cps.append(out_dma(b, slice(0, HALF), bs=b))
                for b in range(B):
                    rd_hi[b].wait_recv()
                    proj(slice(HALF, SQ), bs=b)
                    cps.append(out_dma(2 + b, slice(HALF, SQ), bs=b))
                rd_fwd[0].wait_send()
                rd_fwd[1].wait_send()
                for cp in cps:
                    cp.wait()

            @pl.when(my_pos == 2)
            def _c2():
                rd_hi2 = [
                    pltpu.make_async_remote_copy(
                        src_ref=ctx_buf.at[b, pl.ds(HALF, HALF), :],
                        dst_ref=ctx_buf.at[b, pl.ds(HALF, HALF), :],
                        send_sem=cx_send.at[b],
                        recv_sem=cx_recv.at[2 + b],
                        device_id=(1,), device_id_type=pl.DeviceIdType.MESH,
                    )
                    for b in range(B)
                ]
                cps = []
                rd_hi2[0].wait_recv()
                proj(slice(HALF, SQ), bs=0)
                cps.append(out_dma(0, slice(HALF, SQ), bs=0))
                rd_hi2[1].wait_recv()
                proj(slice(HALF, SQ), bs=1)
                cps.append(out_dma(1, slice(HALF, SQ), bs=1))
                rd_lo[0].wait_recv()
                proj(slice(0, HALF), bs=0)
                cps.append(out_dma(2, slice(0, HALF), bs=0))
                rd_lo[1].wait_recv()
                proj(slice(0, HALF), bs=1)
                cps.append(out_dma(3, slice(0, HALF), bs=1))
                for cp in cps:
                    cp.wait()

    return pl.pallas_call(
        body,
        out_shape=jax.ShapeDtypeStruct((B, SQ, 512), jnp.float32),
        in_specs=[pl.BlockSpec(memory_space=pltpu.VMEM)] * 5,
        out_specs=pl.BlockSpec(memory_space=pltpu.MemorySpace.HBM),
        scratch_shapes=[
            pltpu.VMEM((2, B, SQ, DL), BF),
            pltpu.VMEM((B, SQ, D), BF),
            pltpu.VMEM((B, SQ, 512), jnp.float32),
            pltpu.SemaphoreType.DMA((4,)),
            pltpu.SemaphoreType.DMA((B,)),
            pltpu.SemaphoreType.DMA((B,)),
            pltpu.SemaphoreType.DMA((6,)),
            pltpu.SemaphoreType.DMA((4,)),
        ],
        compiler_params=pltpu.CompilerParams(collective_id=0),
    )(x, Wq, K2, V2, Wo)


# device time: 13028 ns/iter; 1.1012x vs baseline; 1.0051x over previous
import jax
import jax.numpy as jnp
from jax import lax
from jax.experimental import pallas as pl
from jax.experimental.pallas import tpu as pltpu

N_DEV = 4
B = 2
SQ = 256
SKV = 256
HQ = 4
DH = 64
D = HQ * DH
HALF = 128
NEG = -1e9
DL = D + 8
BF = jnp.bfloat16


def kernel(x, Wq, K_ext, V_ext, Wo):
    K2 = K_ext.reshape(B, SKV, D).astype(BF)
    V2 = V_ext.reshape(B, SKV, D).astype(BF)

    def body(x_ref, wq_ref, k_ref, v_ref, wo_ref, out_ref,
             o_buf, ctx_buf, ex_send, ex_recv, cx_send, cx_recv):
        my_pos = lax.axis_index("i")
        left = lax.rem(my_pos + (N_DEV - 1), N_DEV)
        right = lax.rem(my_pos + 1, N_DEV)
        is01 = my_pos <= 1

        barrier_sem = pltpu.get_barrier_semaphore()
        for nbr in (left, right):
            pl.semaphore_signal(
                barrier_sem, inc=1,
                device_id=(nbr,), device_id_type=pl.DeviceIdType.MESH,
            )
        pl.semaphore_wait(barrier_sem, 2)

        def proj(rows, bs=None):
            n = rows.stop - rows.start
            if bs is None:
                c = ctx_buf[:, rows, :].reshape(B * n, D)
                o = jnp.dot(c, wo_ref[...].astype(BF),
                            preferred_element_type=jnp.float32)
                out_ref[:, rows, :] = o.reshape(B, n, 512)
            else:
                c = ctx_buf[bs, rows, :]
                o = jnp.dot(c, wo_ref[...].astype(BF),
                            preferred_element_type=jnp.float32)
                out_ref[bs, rows, :] = o

        def attn_block(qrows, kbh, vbh, bias):
            s = lax.dot_general(
                qrows, kbh, (((1,), (1,)), ((), ())),
                preferred_element_type=jnp.float32,
            )
            w = jnp.exp((s + bias).astype(BF))
            l = jnp.sum(w, axis=1, keepdims=True, dtype=jnp.float32)
            o = jnp.dot(w, vbh, preferred_element_type=jnp.float32)
            return o, l

        def combine_hi(b):
            p = o_buf[0, b, HALF:, :] + o_buf[1, b, HALF:, :]
            for h in range(HQ):
                hs = slice(h * DH, (h + 1) * DH)
                ctx_buf[b, HALF:, hs] = p[:, hs] / p[:, D + h:D + h + 1]

        @pl.when(is01)
        def _producer():
            partner = 1 - my_pos
            rd_ex = [
                pltpu.make_async_remote_copy(
                    src_ref=o_buf.at[0, b, pl.ds(HALF, HALF), :],
                    dst_ref=o_buf.at[1, b, pl.ds(HALF, HALF), :],
                    send_sem=ex_send.at[b], recv_sem=ex_recv.at[b],
                    device_id=(partner,),
                    device_id_type=pl.DeviceIdType.MESH,
                )
                for b in range(B)
            ]
            rd_lo3 = [
                pltpu.make_async_remote_copy(
                    src_ref=ctx_buf.at[b, pl.ds(0, HALF), :],
                    dst_ref=ctx_buf.at[b, pl.ds(0, HALF), :],
                    send_sem=cx_send.at[b], recv_sem=cx_recv.at[b],
                    device_id=(3,), device_id_type=pl.DeviceIdType.MESH,
                )
                for b in range(B)
            ]
            rd_lo1 = [
                pltpu.make_async_remote_copy(
                    src_ref=ctx_buf.at[b, pl.ds(0, HALF), :],
                    dst_ref=ctx_buf.at[b, pl.ds(0, HALF), :],
                    send_sem=cx_send.at[2 + b], recv_sem=cx_recv.at[b],
                    device_id=(1,), device_id_type=pl.DeviceIdType.MESH,
                )
                for b in range(B)
            ]

            @pl.when(my_pos == 0)
            def _p0():
                ti = lax.broadcasted_iota(jnp.int32, (HALF, SKV), 0)
                tj = lax.broadcasted_iota(jnp.int32, (HALF, SKV), 1)
                bias = jnp.where(tj >= ti, 0.0, NEG)
                xhi = x_ref[:, HALF:, :].reshape(B * HALF, 512).astype(BF)
                qhi = (jnp.dot(xhi, wq_ref[...].astype(BF),
                               preferred_element_type=jnp.float32)
                       * 0.125).astype(BF)
                for b in range(B):
                    kb = k_ref[b]
                    vb = v_ref[b]
                    for h in range(HQ):
                        hs = slice(h * DH, (h + 1) * DH)
                        o, l = attn_block(
                            qhi[b * HALF:(b + 1) * HALF, hs],
                            kb[:, hs], vb[:, hs], bias)
                        o_buf[0, b, HALF:, hs] = o.astype(BF)
                        o_buf[0, b, HALF:, D + h:D + h + 1] = l.astype(BF)
                    rd_ex[b].start()

                bias_lo = jnp.where(tj - ti <= 128, 0.0, NEG)
                xlo = x_ref[:, 0:HALF, :].reshape(B * HALF, 512).astype(BF)
                qlo = (jnp.dot(xlo, wq_ref[...].astype(BF),
                               preferred_element_type=jnp.float32)
                       * 0.125).astype(BF)
                for b in range(B):
                    kb = k_ref[b]
                    vb = v_ref[b]
                    for h in range(HQ):
                        hs = slice(h * DH, (h + 1) * DH)
                        o, l = attn_block(
                            qlo[b * HALF:(b + 1) * HALF, hs],
                            kb[:, hs], vb[:, hs], bias_lo)
                        ctx_buf[b, 0:HALF, hs] = (o / l).astype(BF)
                    rd_lo3[b].start()
                    rd_lo1[b].start()

                rd_hi3 = [
                    pltpu.make_async_remote_copy(
                        src_ref=ctx_buf.at[b, pl.ds(HALF, HALF), :],
                        dst_ref=ctx_buf.at[b, pl.ds(HALF, HALF), :],
                        send_sem=cx_send.at[4 + b], recv_sem=cx_recv.at[2 + b],
                        device_id=(3,), device_id_type=pl.DeviceIdType.MESH,
                    )
                    for b in range(B)
                ]
                for b in range(B):
                    rd_ex[b].wait()
                    combine_hi(b)
                    rd_hi3[b].start()
                proj(slice(0, SQ))
                for b in range(B):
                    rd_lo3[b].wait_send()
                    rd_lo1[b].wait_send()
                    rd_hi3[b].wait_send()

            @pl.when(my_pos == 1)
            def _p1():
                ti = lax.broadcasted_iota(jnp.int32, (HALF, HALF), 0)
                tj = lax.broadcasted_iota(jnp.int32, (HALF, HALF), 1)
                bias = jnp.where(tj <= ti, 0.0, NEG)
                xhi = x_ref[:, HALF:, :].reshape(B * HALF, 512).astype(BF)
                qhi = (jnp.dot(xhi, wq_ref[...].astype(BF),
                               preferred_element_type=jnp.float32)
                       * 0.125).astype(BF)
                for b in range(B):
                    kb = k_ref[b, 0:HALF, :]
                    vb = v_ref[b, 0:HALF, :]
                    for h in range(HQ):
                        hs = slice(h * DH, (h + 1) * DH)
                        o, l = attn_block(
                            qhi[b * HALF:(b + 1) * HALF, hs],
                            kb[:, hs], vb[:, hs], bias)
                        o_buf[0, b, HALF:, hs] = o.astype(BF)
                        o_buf[0, b, HALF:, D + h:D + h + 1] = l.astype(BF)
                    rd_ex[b].start()

                rd_hi2 = [
                    pltpu.make_async_remote_copy(
                        src_ref=ctx_buf.at[b, pl.ds(HALF, HALF), :],
                        dst_ref=ctx_buf.at[b, pl.ds(HALF, HALF), :],
                        send_sem=cx_send.at[b],
                        recv_sem=cx_recv.at[2 + b],
                        device_id=(2,), device_id_type=pl.DeviceIdType.MESH,
                    )
                    for b in range(B)
                ]
                for b in range(B):
                    rd_ex[b].wait()
                    combine_hi(b)
                    rd_hi2[b].start()
                    proj(slice(HALF, SQ), bs=b)
                for b in range(B):
                    rd_lo1[b].wait_recv()
                    proj(slice(0, HALF), bs=b)
                rd_hi2[0].wait_send()
                rd_hi2[1].wait_send()

        @pl.when(jnp.logical_not(is01))
        def _consumer():
            rd_lo = [
                pltpu.make_async_remote_copy(
                    src_ref=ctx_buf.at[b, pl.ds(0, HALF), :],
                    dst_ref=ctx_buf.at[b, pl.ds(0, HALF), :],
                    send_sem=cx_send.at[b], recv_sem=cx_recv.at[b],
                    device_id=(left,), device_id_type=pl.DeviceIdType.MESH,
                )
                for b in range(B)
            ]

            @pl.when(my_pos == 3)
            def _c3():
                rd_hi = [
                    pltpu.make_async_remote_copy(
                        src_ref=ctx_buf.at[b, pl.ds(HALF, HALF), :],
                        dst_ref=ctx_buf.at[b, pl.ds(HALF, HALF), :],
                        send_sem=cx_send.at[4 + b], recv_sem=cx_recv.at[2 + b],
                        device_id=(0,), device_id_type=pl.DeviceIdType.MESH,
                    )
                    for b in range(B)
                ]
                rd_fwd = [
                    pltpu.make_async_remote_copy(
                        src_ref=ctx_buf.at[b, pl.ds(0, HALF), :],
                        dst_ref=ctx_buf.at[b, pl.ds(0, HALF), :],
                        send_sem=cx_send.at[b], recv_sem=cx_recv.at[b],
                        device_id=(2,), device_id_type=pl.DeviceIdType.MESH,
                    )
                    for b in range(B)
                ]
                for b in range(B):
                    rd_lo[b].wait_recv()
                    rd_fwd[b].start()
                    proj(slice(0, HALF), bs=b)
                for b in range(B):
                    rd_hi[b].wait_recv()
                    proj(slice(HALF, SQ), bs=b)
                rd_fwd[0].wait_send()
                rd_fwd[1].wait_send()

            @pl.when(my_pos == 2)
            def _c2():
                rd_hi2 = [
                    pltpu.make_async_remote_copy(
                        src_ref=ctx_buf.at[b, pl.ds(HALF, HALF), :],
                        dst_ref=ctx_buf.at[b, pl.ds(HALF, HALF), :],
                        send_sem=cx_send.at[b],
                        recv_sem=cx_recv.at[2 + b],
                        device_id=(1,), device_id_type=pl.DeviceIdType.MESH,
                    )
                    for b in range(B)
                ]
                rd_hi2[0].wait_recv()
                proj(slice(HALF, SQ), bs=0)
                rd_hi2[1].wait_recv()
                proj(slice(HALF, SQ), bs=1)
                rd_lo[0].wait_recv()
                proj(slice(0, HALF), bs=0)
                rd_lo[1].wait_recv()
                proj(slice(0, HALF), bs=1)

    return pl.pallas_call(
        body,
        out_shape=jax.ShapeDtypeStruct((B, SQ, 512), jnp.float32),
        in_specs=[pl.BlockSpec(memory_space=pltpu.VMEM)] * 5,
        out_specs=pl.BlockSpec(memory_space=pltpu.VMEM),
        scratch_shapes=[
            pltpu.VMEM((2, B, SQ, DL), BF),
            pltpu.VMEM((B, SQ, D), BF),
            pltpu.SemaphoreType.DMA((B,)),
            pltpu.SemaphoreType.DMA((B,)),
            pltpu.SemaphoreType.DMA((6,)),
            pltpu.SemaphoreType.DMA((4,)),
        ],
        compiler_params=pltpu.CompilerParams(collective_id=0),
    )(x, Wq, K2, V2, Wo)
